# baseline (speedup 1.0000x reference)
import functools

import jax
import jax.numpy as jnp
from jax import lax
from jax.experimental import pallas as pl
from jax.experimental.pallas import tpu as pltpu
from jax.experimental.pallas import tpu_sc as plsc


def _make_sc_gather(total, dim, chunkp):
    info = plsc.get_sparse_core_info()
    nc, ns = info.num_cores, info.num_subcores
    nw = nc * ns
    half = total // 2
    per_w = half // nw
    n_iter = per_w // chunkp
    mesh = plsc.VectorSubcoreMesh(core_axis_name="c", subcore_axis_name="s")

    @functools.partial(
        pl.kernel,
        mesh=mesh,
        out_type=jax.ShapeDtypeStruct((half, 2 * dim), jnp.float32),
        compiler_params=pltpu.CompilerParams(needs_layout_passes=False),
        scratch_types=[
            pltpu.VMEM((chunkp,), jnp.int32),
            pltpu.VMEM((chunkp,), jnp.int32),
            pltpu.VMEM((chunkp,), jnp.int32),
            pltpu.VMEM((chunkp,), jnp.int32),
            pltpu.VMEM((chunkp, 2 * dim), jnp.float32),
            pltpu.VMEM((chunkp, 2 * dim), jnp.float32),
            pltpu.VMEM((chunkp, 2 * dim), jnp.float32),
            pltpu.SemaphoreType.DMA,
            pltpu.SemaphoreType.DMA,
        ],
    )
    def gather(idx2_hbm, off_hbm, table2_hbm, emb2_hbm,
               idx2a, offa, idx2b, offb, rowsa, rowsb, outv, sema, semb):
        wid = lax.axis_index("s") * nc + lax.axis_index("c")
        w_base = wid * per_w
        iota = lax.iota(jnp.int32, 16)

        def body(i, carry):
            base = w_base + i * chunkp
            pltpu.sync_copy(idx2_hbm.at[pl.ds(base, chunkp)], idx2a)
            pltpu.sync_copy(off_hbm.at[pl.ds(base, chunkp)], offa)
            pltpu.sync_copy(idx2_hbm.at[pl.ds(base + half, chunkp)], idx2b)
            pltpu.sync_copy(off_hbm.at[pl.ds(base + half, chunkp)], offb)
            cpa = pltpu.async_copy(table2_hbm.at[idx2a], rowsa, sema)
            cpb = pltpu.async_copy(table2_hbm.at[idx2b], rowsb, semb)
            cpa.wait()
            cpb.wait()

            def sel(g, carry2):
                rowv = iota + g * 16
                oa = offa[pl.ds(g * 16, 16)]
                ob = offb[pl.ds(g * 16, 16)]
                for c in range(dim):
                    cv = jnp.full((16,), c, jnp.int32)
                    va = plsc.load_gather(rowsa, [rowv, oa + c])
                    plsc.store_scatter(outv, [rowv, cv], va)
                    vb = plsc.load_gather(rowsb, [rowv, ob + c])
                    plsc.store_scatter(outv, [rowv, cv + dim], vb)
                return carry2

            lax.fori_loop(0, chunkp // 16, sel, 0)
            pltpu.sync_copy(outv, emb2_hbm.at[pl.ds(base, chunkp)])
            return carry

        lax.fori_loop(0, n_iter, body, 0)

    return gather


def _mm_body(emb_ref, w_ref, b_ref, out_ref):
    out_ref[...] = (
        jnp.dot(emb_ref[...], w_ref[0], preferred_element_type=jnp.float32)
        + b_ref[...]
    )


def _project(emb2, w3, b2, block_m):
    half, dim2 = emb2.shape
    out_dim = w3.shape[2]
    g2 = half // block_m
    return pl.pallas_call(
        _mm_body,
        grid=(g2, 2),
        in_specs=[
            pl.BlockSpec((block_m, dim2), lambda i, s: (i, 0)),
            pl.BlockSpec((1, dim2, out_dim), lambda i, s: (s, 0, 0)),
            pl.BlockSpec((1, out_dim), lambda i, s: (0, 0)),
        ],
        out_specs=pl.BlockSpec((block_m, out_dim), lambda i, s: (s * g2 + i, 0)),
        out_shape=jax.ShapeDtypeStruct((2 * half, out_dim), jnp.float32),
    )(emb2, w3, b2)


def kernel(x, table, W, b):
    batch, hist = x.shape
    vocab, dim = table.shape
    out_dim = W.shape[0]
    total = batch * hist

    idx = x.reshape(-1).astype(jnp.int32)
    idx2 = idx >> 1
    off = (idx & 1) << 6
    table2 = table.reshape(vocab // 2, 2 * dim)

    scale = jnp.sqrt(jnp.float32(dim))
    wt = (W * scale).T
    zero = jnp.zeros((dim, out_dim), jnp.float32)
    w_lo = jnp.concatenate([wt, zero], axis=0)
    w_hi = jnp.concatenate([zero, wt], axis=0)
    w3 = jnp.stack([w_lo, w_hi], axis=0)
    b2 = b.reshape(1, out_dim)

    gather = _make_sc_gather(total, dim, chunkp=256)
    emb2 = gather(idx2, off, table2)

    out = _project(emb2, w3, b2, block_m=2048)
    return out.reshape(batch, hist, out_dim)


# speedup vs baseline: 1.5805x; 1.5805x over previous
import functools

import jax
import jax.numpy as jnp
from jax import lax
from jax.experimental import pallas as pl
from jax.experimental.pallas import tpu as pltpu
from jax.experimental.pallas import tpu_sc as plsc


def _make_sc_gather(total, dim, chunkp):
    info = plsc.get_sparse_core_info()
    nc, ns = info.num_cores, info.num_subcores
    nw = nc * ns
    half = total // 2
    per_w = half // nw
    n_iter = per_w // chunkp
    mesh = plsc.VectorSubcoreMesh(core_axis_name="c", subcore_axis_name="s")

    @functools.partial(
        pl.kernel,
        mesh=mesh,
        out_type=jax.ShapeDtypeStruct((half, 2 * dim), jnp.float32),
        compiler_params=pltpu.CompilerParams(needs_layout_passes=False),
        scratch_types=[
            pltpu.VMEM((chunkp,), jnp.int32),
            pltpu.VMEM((chunkp,), jnp.int32),
            pltpu.VMEM((chunkp,), jnp.int32),
            pltpu.VMEM((chunkp,), jnp.int32),
            pltpu.VMEM((chunkp, 2 * dim), jnp.float32),
            pltpu.VMEM((chunkp, 2 * dim), jnp.float32),
            pltpu.VMEM((chunkp, 2 * dim), jnp.float32),
            pltpu.SemaphoreType.DMA,
            pltpu.SemaphoreType.DMA,
        ],
    )
    def gather(idx2_hbm, off_hbm, table2_hbm, emb2_hbm,
               idx2a, idx2b, offa, offb, rowsa, rowsb, outv, sema, semb):
        wid = lax.axis_index("s") * nc + lax.axis_index("c")
        w_base = wid * per_w

        def body(i, carry):
            base = w_base + i * chunkp
            pltpu.sync_copy(idx2_hbm.at[pl.ds(base, chunkp)], idx2a)
            pltpu.sync_copy(idx2_hbm.at[pl.ds(base + half, chunkp)], idx2b)
            pltpu.sync_copy(off_hbm.at[pl.ds(base, chunkp)], offa)
            pltpu.sync_copy(off_hbm.at[pl.ds(base + half, chunkp)], offb)
            cpa = pltpu.async_copy(table2_hbm.at[idx2a], rowsa, sema)
            cpb = pltpu.async_copy(table2_hbm.at[idx2b], rowsb, semb)
            cpa.wait()
            cpb.wait()

            def sel(g, carry2):
                oavec = offa[pl.ds(g * 16, 16)]
                obvec = offb[pl.ds(g * 16, 16)]
                for j in range(16):
                    r = g * 16 + j
                    oa = oavec[j]
                    ob = obvec[j]
                    for k in range(dim // 16):
                        outv[r, pl.ds(k * 16, 16)] = rowsa[r, pl.ds(oa + k * 16, 16)]
                        outv[r, pl.ds(dim + k * 16, 16)] = rowsb[r, pl.ds(ob + k * 16, 16)]
                return carry2

            lax.fori_loop(0, chunkp // 16, sel, 0)
            pltpu.sync_copy(outv, emb2_hbm.at[pl.ds(base, chunkp)])
            return carry

        lax.fori_loop(0, n_iter, body, 0)

    return gather


def _mm_body(emb_ref, w_ref, b_ref, out_ref):
    out_ref[...] = (
        jnp.dot(emb_ref[...], w_ref[0], preferred_element_type=jnp.float32)
        + b_ref[...]
    )


def _project(emb2, w3, b2, block_m):
    half, dim2 = emb2.shape
    out_dim = w3.shape[2]
    g2 = half // block_m
    return pl.pallas_call(
        _mm_body,
        grid=(g2, 2),
        in_specs=[
            pl.BlockSpec((block_m, dim2), lambda i, s: (i, 0)),
            pl.BlockSpec((1, dim2, out_dim), lambda i, s: (s, 0, 0)),
            pl.BlockSpec((1, out_dim), lambda i, s: (0, 0)),
        ],
        out_specs=pl.BlockSpec((block_m, out_dim), lambda i, s: (s * g2 + i, 0)),
        out_shape=jax.ShapeDtypeStruct((2 * half, out_dim), jnp.float32),
    )(emb2, w3, b2)


def kernel(x, table, W, b):
    batch, hist = x.shape
    vocab, dim = table.shape
    out_dim = W.shape[0]
    total = batch * hist

    idx = x.reshape(-1).astype(jnp.int32)
    idx2 = idx >> 1
    off = (idx & 1) << 6
    table2 = table.reshape(vocab // 2, 2 * dim)

    scale = jnp.sqrt(jnp.float32(dim))
    wt = (W * scale).T
    zero = jnp.zeros((dim, out_dim), jnp.float32)
    w_lo = jnp.concatenate([wt, zero], axis=0)
    w_hi = jnp.concatenate([zero, wt], axis=0)
    w3 = jnp.stack([w_lo, w_hi], axis=0)
    b2 = b.reshape(1, out_dim)

    gather = _make_sc_gather(total, dim, chunkp=256)
    emb2 = gather(idx2, off, table2)

    out = _project(emb2, w3, b2, block_m=2048)
    return out.reshape(batch, hist, out_dim)


# speedup vs baseline: 1.8058x; 1.1426x over previous
import functools

import jax
import jax.numpy as jnp
from jax import lax
from jax.experimental import pallas as pl
from jax.experimental.pallas import tpu as pltpu
from jax.experimental.pallas import tpu_sc as plsc


def _make_sc_gather(total, dim, chunkp, s_chunks):
    info = plsc.get_sparse_core_info()
    nc, ns = info.num_cores, info.num_subcores
    nw = nc * ns
    half = total // 2
    per_w = half // nw
    n_iter = per_w // chunkp
    n_super = n_iter // s_chunks
    assert n_iter % s_chunks == 0 and n_iter % 2 == 0
    scm = s_chunks * chunkp
    dim2 = 2 * dim
    mesh = plsc.VectorSubcoreMesh(core_axis_name="c", subcore_axis_name="s")

    @functools.partial(
        pl.kernel,
        mesh=mesh,
        out_type=jax.ShapeDtypeStruct((half, dim2), jnp.float32),
        compiler_params=pltpu.CompilerParams(needs_layout_passes=False),
        scratch_types=[
            pltpu.VMEM((2 * scm,), jnp.int32),
            pltpu.VMEM((2 * scm,), jnp.int32),
            pltpu.VMEM((2 * scm,), jnp.int32),
            pltpu.VMEM((2 * scm,), jnp.int32),
            pltpu.VMEM((2, chunkp, dim2), jnp.float32),
            pltpu.VMEM((2, chunkp, dim2), jnp.float32),
            pltpu.VMEM((2, chunkp, dim2), jnp.float32),
            pltpu.SemaphoreType.DMA,
            pltpu.SemaphoreType.DMA,
            pltpu.SemaphoreType.DMA,
            pltpu.SemaphoreType.DMA,
            pltpu.SemaphoreType.DMA,
            pltpu.SemaphoreType.DMA,
            pltpu.SemaphoreType.DMA,
        ],
    )
    def gather(i2a_hbm, i2b_hbm, ofa_hbm, ofb_hbm, table2_hbm, emb2_hbm,
               ia, ib, oab, obb, rowsa, rowsb, outv,
               sga0, sga1, sgb0, sgb1, swb0, swb1, semm):
        wid = lax.axis_index("s") * nc + lax.axis_index("c")
        w_base = wid * per_w
        sga = (sga0, sga1)
        sgb = (sgb0, sgb1)
        swb = (swb0, swb1)

        def mpos(i):
            return ((i // s_chunks) % 2) * scm + (i % s_chunks) * chunkp

        def meta_fire(s):
            reg = (s % 2) * scm
            base = w_base + s * scm
            pltpu.async_copy(i2a_hbm.at[pl.ds(base, scm)], ia.at[pl.ds(reg, scm)], semm)
            pltpu.async_copy(i2b_hbm.at[pl.ds(base, scm)], ib.at[pl.ds(reg, scm)], semm)
            pltpu.async_copy(ofa_hbm.at[pl.ds(base, scm)], oab.at[pl.ds(reg, scm)], semm)
            pltpu.async_copy(ofb_hbm.at[pl.ds(base, scm)], obb.at[pl.ds(reg, scm)], semm)

        def meta_wait(s):
            reg = (s % 2) * scm
            base = w_base + s * scm
            pltpu.make_async_copy(i2a_hbm.at[pl.ds(base, scm)], ia.at[pl.ds(reg, scm)], semm).wait()
            pltpu.make_async_copy(i2b_hbm.at[pl.ds(base, scm)], ib.at[pl.ds(reg, scm)], semm).wait()
            pltpu.make_async_copy(ofa_hbm.at[pl.ds(base, scm)], oab.at[pl.ds(reg, scm)], semm).wait()
            pltpu.make_async_copy(ofb_hbm.at[pl.ds(base, scm)], obb.at[pl.ds(reg, scm)], semm).wait()

        def gather_fire(b, j):
            @pl.when(jnp.logical_and(j % s_chunks == 0, j > 0))
            def _():
                meta_wait(j // s_chunks)
            mp = mpos(j)
            pltpu.async_copy(table2_hbm.at[ia.at[pl.ds(mp, chunkp)]], rowsa.at[b], sga[b])
            pltpu.async_copy(table2_hbm.at[ib.at[pl.ds(mp, chunkp)]], rowsb.at[b], sgb[b])

        def consume(b, i):
            pltpu.make_async_copy(
                table2_hbm.at[ia.at[pl.ds(0, chunkp)]], rowsa.at[b], sga[b]).wait()
            pltpu.make_async_copy(
                table2_hbm.at[ib.at[pl.ds(0, chunkp)]], rowsb.at[b], sgb[b]).wait()

            @pl.when(i >= 2)
            def _():
                pltpu.make_async_copy(
                    outv.at[b], emb2_hbm.at[pl.ds(w_base, chunkp)], swb[b]).wait()

            mp = mpos(i)

            def sel(g, c2):
                oavec = oab[pl.ds(mp + g * 16, 16)]
                obvec = obb[pl.ds(mp + g * 16, 16)]
                for j16 in range(16):
                    r = g * 16 + j16
                    oa = oavec[j16]
                    ob = obvec[j16]
                    for k in range(dim // 16):
                        outv[b, r, pl.ds(k * 16, 16)] = rowsa[b, r, pl.ds(oa + k * 16, 16)]
                        outv[b, r, pl.ds(dim + k * 16, 16)] = rowsb[b, r, pl.ds(ob + k * 16, 16)]
                return c2

            lax.fori_loop(0, chunkp // 16, sel, 0)
            pltpu.async_copy(
                outv.at[b], emb2_hbm.at[pl.ds(w_base + i * chunkp, chunkp)], swb[b])

            @pl.when(jnp.logical_and(i % s_chunks == 0, i + s_chunks < n_iter))
            def _():
                meta_fire(i // s_chunks + 1)

            @pl.when(i + 2 < n_iter)
            def _():
                gather_fire(b, i + 2)

        meta_fire(0)
        meta_wait(0)
        gather_fire(0, 0)
        gather_fire(1, 1)

        def body(k, carry):
            for b in range(2):
                consume(b, 2 * k + b)
            return carry

        lax.fori_loop(0, n_iter // 2, body, 0)
        pltpu.make_async_copy(
            outv.at[0], emb2_hbm.at[pl.ds(w_base, chunkp)], swb0).wait()
        pltpu.make_async_copy(
            outv.at[1], emb2_hbm.at[pl.ds(w_base, chunkp)], swb1).wait()

    return gather


def _mm_body(emb_ref, w_ref, b_ref, out_ref):
    out_ref[...] = (
        jnp.dot(emb_ref[...], w_ref[0], preferred_element_type=jnp.float32)
        + b_ref[...]
    )


def _project(emb2, w3, b2, block_m):
    half, dim2 = emb2.shape
    out_dim = w3.shape[2]
    g2 = half // block_m
    return pl.pallas_call(
        _mm_body,
        grid=(g2, 2),
        in_specs=[
            pl.BlockSpec((block_m, dim2), lambda i, s: (i, 0)),
            pl.BlockSpec((1, dim2, out_dim), lambda i, s: (s, 0, 0)),
            pl.BlockSpec((1, out_dim), lambda i, s: (0, 0)),
        ],
        out_specs=pl.BlockSpec((block_m, out_dim), lambda i, s: (s * g2 + i, 0)),
        out_shape=jax.ShapeDtypeStruct((2 * half, out_dim), jnp.float32),
    )(emb2, w3, b2)


def kernel(x, table, W, b):
    batch, hist = x.shape
    vocab, dim = table.shape
    out_dim = W.shape[0]
    total = batch * hist
    half = total // 2

    idx = x.reshape(-1).astype(jnp.int32)
    idx2 = idx >> 1
    off = (idx & 1) << 6
    table2 = table.reshape(vocab // 2, 2 * dim)

    scale = jnp.sqrt(jnp.float32(dim))
    wt = (W * scale).T
    zero = jnp.zeros((dim, out_dim), jnp.float32)
    w_lo = jnp.concatenate([wt, zero], axis=0)
    w_hi = jnp.concatenate([zero, wt], axis=0)
    w3 = jnp.stack([w_lo, w_hi], axis=0)
    b2 = b.reshape(1, out_dim)

    gather = _make_sc_gather(total, dim, chunkp=128, s_chunks=10)
    emb2 = gather(idx2[:half], idx2[half:], off[:half], off[half:], table2)

    out = _project(emb2, w3, b2, block_m=2048)
    return out.reshape(batch, hist, out_dim)
